# baseline (speedup 1.0000x reference)
import functools

import jax
import jax.numpy as jnp
from jax import lax
from jax.experimental import pallas as pl
from jax.experimental.pallas import tpu as pltpu
from jax.experimental.pallas import tpu_sc as plsc

B = 16384
NSTIM = 21
NPAIR = NSTIM * NSTIM
NPAIR_PAD = 448
L = 16
NC, NS = 2, 16
NW = NC * NS
BPW = B // NW
NGROUPS = BPW // L

_PAIRS = [(i, j) for i in range(8) for j in range(8) if j != i]


def _sqrt_f32(x):
    i = plsc.bitcast(x, jnp.int32)
    i = jnp.int32(0x5F3759DF) - jnp.right_shift(i, 1)
    y = plsc.bitcast(i, jnp.float32)
    for _ in range(3):
        y = y * (1.5 - 0.5 * x * y * y)
    return jnp.where(x > 0.0, x * y, 0.0)


def _splat(v, dtype=jnp.int32):
    return jnp.full((L,), v, dtype)


def _sc_body(g2_hbm, g8_hbm, tab_hbm, out1_hbm, out2_hbm,
             tab_v, s_v, idx2_v, idx8_v, out1_v, out2_v):
    wid = lax.axis_index("s") * NC + lax.axis_index("c")
    base = wid * BPW
    pltpu.sync_copy(tab_hbm, tab_v)
    pltpu.sync_copy(g2_hbm.at[pl.ds(base, BPW)], idx2_v)
    pltpu.sync_copy(g8_hbm.at[pl.ds(base, BPW)], idx8_v)

    def build_pairs(i, carry):
        p = jnp.minimum(i * L + lax.iota(jnp.int32, L), NPAIR - 1)
        q = jnp.right_shift(p * 3121, 16)
        r = p - q * NSTIM
        q3, r3 = q * 3, r * 3
        d2 = None
        for d in range(3):
            diff = (plsc.load_gather(tab_v, [q3 + d])
                    - plsc.load_gather(tab_v, [r3 + d]))
            d2 = diff * diff if d2 is None else d2 + diff * diff
        s_v[pl.ds(i * L, L)] = jnp.exp(-10.0 * _sqrt_f32(d2)) + 0.001
        return carry

    lax.fori_loop(0, NPAIR_PAD // L, build_pairs, 0)

    def group(g, carry):
        rows = g * L + lax.iota(jnp.int32, L)
        q = plsc.load_gather(idx2_v, [rows, _splat(0)]) * NSTIM
        s1 = plsc.load_gather(
            s_v, [q + plsc.load_gather(idx2_v, [rows, _splat(1)])])
        s2 = plsc.load_gather(
            s_v, [q + plsc.load_gather(idx2_v, [rows, _splat(2)])])
        inv = 1.0 / (s1 + s2)
        plsc.store_scatter(out1_v, [rows, _splat(0)], s1 * inv)
        plsc.store_scatter(out1_v, [rows, _splat(1)], s2 * inv)
        q = plsc.load_gather(idx8_v, [rows, _splat(0)]) * NSTIM
        s = [plsc.load_gather(
                s_v, [q + plsc.load_gather(idx8_v, [rows, _splat(j + 1)])])
             for j in range(8)]
        tot = s[0]
        for j in range(1, 8):
            tot = tot + s[j]
        invt = 1.0 / tot
        a = [(s[i] * invt) / (tot - s[i]) for i in range(8)]
        for k, (i, j) in enumerate(_PAIRS):
            plsc.store_scatter(out2_v, [rows, _splat(k)], a[i] * s[j])
        return carry

    lax.fori_loop(0, NGROUPS, group, 0)

    pltpu.sync_copy(out1_v, out1_hbm.at[pl.ds(base, BPW)])
    pltpu.sync_copy(out2_v, out2_hbm.at[pl.ds(base, BPW)])


@functools.cache
def _build():
    mesh = plsc.VectorSubcoreMesh(
        core_axis_name="c", subcore_axis_name="s",
        num_cores=NC, num_subcores=NS)
    return pl.kernel(
        _sc_body,
        out_type=(jax.ShapeDtypeStruct((B, 2), jnp.float32),
                  jax.ShapeDtypeStruct((B, 56), jnp.float32)),
        mesh=mesh,
        compiler_params=pltpu.CompilerParams(
            needs_layout_passes=False, use_tc_tiling_on_sc=False),
        scratch_types=[
            pltpu.VMEM((64,), jnp.float32),
            pltpu.VMEM((NPAIR_PAD,), jnp.float32),
            pltpu.VMEM((BPW, 3), jnp.int32),
            pltpu.VMEM((BPW, 9), jnp.int32),
            pltpu.VMEM((BPW, 2), jnp.float32),
            pltpu.VMEM((BPW, 56), jnp.float32),
        ],
    )


def kernel(given2rank1_stimulus_set, given8rank2_stimulus_set, percept_table):
    tab_flat = jnp.pad(percept_table.reshape(-1), (0, 64 - 3 * NSTIM))
    return _build()(given2rank1_stimulus_set, given8rank2_stimulus_set,
                    tab_flat)


# speedup vs baseline: 1.1464x; 1.1464x over previous
import functools

import jax
import jax.numpy as jnp
from jax import lax
from jax.experimental import pallas as pl
from jax.experimental.pallas import tpu as pltpu
from jax.experimental.pallas import tpu_sc as plsc

B = 16384
NSTIM = 21
NPAIR = NSTIM * NSTIM
NPAIR_PAD = 448
L = 16
NC, NS = 2, 16
NW = NC * NS
BPW = B // NW
CHUNK = 128
NCHUNKS = BPW // CHUNK
NGROUPS = CHUNK // L

_PAIRS = [(i, j) for i in range(8) for j in range(8) if j != i]


def _sqrt_f32(x):
    i = plsc.bitcast(x, jnp.int32)
    i = jnp.int32(0x5F3759DF) - jnp.right_shift(i, 1)
    y = plsc.bitcast(i, jnp.float32)
    for _ in range(3):
        y = y * (1.5 - 0.5 * x * y * y)
    return jnp.where(x > 0.0, x * y, 0.0)


def _splat(v, dtype=jnp.int32):
    return jnp.full((L,), v, dtype)


def _sc_body(g2_hbm, g8_hbm, tab_hbm, out1_hbm, out2_hbm,
             tab_v, s_v, idx2_v, idx8_v, out1_v, out2_v):
    wid = lax.axis_index("s") * NC + lax.axis_index("c")
    base = wid * BPW
    pltpu.sync_copy(tab_hbm, tab_v)

    def build_pairs(i, carry):
        p = jnp.minimum(i * L + lax.iota(jnp.int32, L), NPAIR - 1)
        q = jnp.right_shift(p * 3121, 16)
        r = p - q * NSTIM
        d2 = None
        for d in range(3):
            diff = (plsc.load_gather(tab_v, [q, _splat(d)])
                    - plsc.load_gather(tab_v, [r, _splat(d)]))
            d2 = diff * diff if d2 is None else d2 + diff * diff
        s_v[pl.ds(i * L, L)] = jnp.exp(-10.0 * _sqrt_f32(d2)) + 0.001
        return carry

    lax.fori_loop(0, NPAIR_PAD // L, build_pairs, 0)

    def group(g, carry):
        rows = g * L + lax.iota(jnp.int32, L)
        q = plsc.load_gather(idx2_v, [rows, _splat(0)]) * NSTIM
        s1 = plsc.load_gather(
            s_v, [q + plsc.load_gather(idx2_v, [rows, _splat(1)])])
        s2 = plsc.load_gather(
            s_v, [q + plsc.load_gather(idx2_v, [rows, _splat(2)])])
        inv = 1.0 / (s1 + s2)
        plsc.store_scatter(out1_v, [rows, _splat(0)], s1 * inv)
        plsc.store_scatter(out1_v, [rows, _splat(1)], s2 * inv)
        q = plsc.load_gather(idx8_v, [rows, _splat(0)]) * NSTIM
        s = [plsc.load_gather(
                s_v, [q + plsc.load_gather(idx8_v, [rows, _splat(j + 1)])])
             for j in range(8)]
        tot = s[0]
        for j in range(1, 8):
            tot = tot + s[j]
        invt = 1.0 / tot
        a = [(s[i] * invt) / (tot - s[i]) for i in range(8)]
        for k, (i, j) in enumerate(_PAIRS):
            plsc.store_scatter(out2_v, [rows, _splat(k)], a[i] * s[j])
        return carry

    def chunk(c, carry):
        cb = base + c * CHUNK
        pltpu.sync_copy(g2_hbm.at[pl.ds(cb, CHUNK)], idx2_v)
        pltpu.sync_copy(g8_hbm.at[pl.ds(cb, CHUNK)], idx8_v)
        lax.fori_loop(0, NGROUPS, group, 0)
        pltpu.sync_copy(out1_v, out1_hbm.at[pl.ds(cb, CHUNK)])
        pltpu.sync_copy(out2_v, out2_hbm.at[pl.ds(cb, CHUNK)])
        return carry

    lax.fori_loop(0, NCHUNKS, chunk, 0)


@functools.cache
def _build():
    mesh = plsc.VectorSubcoreMesh(
        core_axis_name="c", subcore_axis_name="s",
        num_cores=NC, num_subcores=NS)
    return pl.kernel(
        _sc_body,
        out_type=(jax.ShapeDtypeStruct((B, 2), jnp.float32),
                  jax.ShapeDtypeStruct((B, 56), jnp.float32)),
        mesh=mesh,
        compiler_params=pltpu.CompilerParams(
            needs_layout_passes=False, use_tc_tiling_on_sc=True),
        scratch_types=[
            pltpu.VMEM((NSTIM, 3), jnp.float32),
            pltpu.VMEM((NPAIR_PAD,), jnp.float32),
            pltpu.VMEM((CHUNK, 3), jnp.int32),
            pltpu.VMEM((CHUNK, 9), jnp.int32),
            pltpu.VMEM((CHUNK, 2), jnp.float32),
            pltpu.VMEM((CHUNK, 56), jnp.float32),
        ],
    )


def kernel(given2rank1_stimulus_set, given8rank2_stimulus_set, percept_table):
    return _build()(given2rank1_stimulus_set, given8rank2_stimulus_set,
                    percept_table)


# speedup vs baseline: 1.3948x; 1.2167x over previous
import functools

import jax
import jax.numpy as jnp
from jax import lax
from jax.experimental import pallas as pl
from jax.experimental.pallas import tpu as pltpu
from jax.experimental.pallas import tpu_sc as plsc

B = 16384
NSTIM = 21
NPAIR = NSTIM * NSTIM
NPAIR_PAD = 448
L = 16
NC, NS = 2, 16
NW = NC * NS
BPW = B // NW
CHUNK = 64
NCHUNKS = BPW // CHUNK
NGROUPS = CHUNK // L

_PAIRS = [(i, j) for i in range(8) for j in range(8) if j != i]


def _sqrt_f32(x):
    i = plsc.bitcast(x, jnp.int32)
    i = jnp.int32(0x5F3759DF) - jnp.right_shift(i, 1)
    y = plsc.bitcast(i, jnp.float32)
    for _ in range(3):
        y = y * (1.5 - 0.5 * x * y * y)
    return jnp.where(x > 0.0, x * y, 0.0)


def _splat(v, dtype=jnp.int32):
    return jnp.full((L,), v, dtype)


def _sc_body(g2_hbm, g8_hbm, tab_hbm, out1_hbm, out2_hbm,
             tab_v, s_v, idx2_v, idx8_v, out1_v, out2_v,
             sem_in2, sem_in8, sem_o1, sem_o2):
    wid = lax.axis_index("s") * NC + lax.axis_index("c")
    base = wid * BPW
    pltpu.sync_copy(tab_hbm, tab_v)

    def in_copy(c):
        cb = base + c * CHUNK
        b = c % 2
        return (pltpu.make_async_copy(
                    g2_hbm.at[pl.ds(cb, CHUNK)], idx2_v.at[b], sem_in2.at[b]),
                pltpu.make_async_copy(
                    g8_hbm.at[pl.ds(cb, CHUNK)], idx8_v.at[b], sem_in8.at[b]))

    def out_copy(c):
        cb = base + c * CHUNK
        b = c % 2
        return (pltpu.make_async_copy(
                    out1_v.at[b], out1_hbm.at[pl.ds(cb, CHUNK)], sem_o1.at[b]),
                pltpu.make_async_copy(
                    out2_v.at[b], out2_hbm.at[pl.ds(cb, CHUNK)], sem_o2.at[b]))

    for cp in in_copy(0):
        cp.start()

    def build_pairs(i, carry):
        p = jnp.minimum(i * L + lax.iota(jnp.int32, L), NPAIR - 1)
        q = jnp.right_shift(p * 3121, 16)
        r = p - q * NSTIM
        d2 = None
        for d in range(3):
            diff = (plsc.load_gather(tab_v, [q, _splat(d)])
                    - plsc.load_gather(tab_v, [r, _splat(d)]))
            d2 = diff * diff if d2 is None else d2 + diff * diff
        s_v[pl.ds(i * L, L)] = jnp.exp(-10.0 * _sqrt_f32(d2)) + 0.001
        return carry

    lax.fori_loop(0, NPAIR_PAD // L, build_pairs, 0)

    def make_group(i2_v, i8_v, o1_v, o2_v):
        def group(g, carry):
            rows = g * L + lax.iota(jnp.int32, L)
            q = plsc.load_gather(i2_v, [rows, _splat(0)]) * NSTIM
            s1 = plsc.load_gather(
                s_v, [q + plsc.load_gather(i2_v, [rows, _splat(1)])])
            s2 = plsc.load_gather(
                s_v, [q + plsc.load_gather(i2_v, [rows, _splat(2)])])
            inv = 1.0 / (s1 + s2)
            plsc.store_scatter(o1_v, [rows, _splat(0)], s1 * inv)
            plsc.store_scatter(o1_v, [rows, _splat(1)], s2 * inv)
            q = plsc.load_gather(i8_v, [rows, _splat(0)]) * NSTIM
            s = [plsc.load_gather(
                    s_v, [q + plsc.load_gather(i8_v, [rows, _splat(j + 1)])])
                 for j in range(8)]
            tot = s[0]
            for j in range(1, 8):
                tot = tot + s[j]
            invt = 1.0 / tot
            a = [(s[i] * invt) / (tot - s[i]) for i in range(8)]
            for k, (i, j) in enumerate(_PAIRS):
                plsc.store_scatter(o2_v, [rows, _splat(k)], a[i] * s[j])
            return carry
        return group

    for c in range(NCHUNKS):
        b = c % 2
        if c + 1 < NCHUNKS:
            for cp in in_copy(c + 1):
                cp.start()
        for cp in in_copy(c):
            cp.wait()
        if c >= 2:
            for cp in out_copy(c - 2):
                cp.wait()
        lax.fori_loop(0, NGROUPS,
                      make_group(idx2_v.at[b], idx8_v.at[b], out1_v.at[b], out2_v.at[b]),
                      0)
        for cp in out_copy(c):
            cp.start()

    for c in (NCHUNKS - 2, NCHUNKS - 1):
        for cp in out_copy(c):
            cp.wait()


@functools.cache
def _build():
    mesh = plsc.VectorSubcoreMesh(
        core_axis_name="c", subcore_axis_name="s",
        num_cores=NC, num_subcores=NS)
    return pl.kernel(
        _sc_body,
        out_type=(jax.ShapeDtypeStruct((B, 2), jnp.float32),
                  jax.ShapeDtypeStruct((B, 56), jnp.float32)),
        mesh=mesh,
        compiler_params=pltpu.CompilerParams(
            needs_layout_passes=False, use_tc_tiling_on_sc=True),
        scratch_types=[
            pltpu.VMEM((NSTIM, 3), jnp.float32),
            pltpu.VMEM((NPAIR_PAD,), jnp.float32),
            pltpu.VMEM((2, CHUNK, 3), jnp.int32),
            pltpu.VMEM((2, CHUNK, 9), jnp.int32),
            pltpu.VMEM((2, CHUNK, 2), jnp.float32),
            pltpu.VMEM((2, CHUNK, 56), jnp.float32),
            pltpu.SemaphoreType.DMA((2,)),
            pltpu.SemaphoreType.DMA((2,)),
            pltpu.SemaphoreType.DMA((2,)),
            pltpu.SemaphoreType.DMA((2,)),
        ],
    )


def kernel(given2rank1_stimulus_set, given8rank2_stimulus_set, percept_table):
    return _build()(given2rank1_stimulus_set, given8rank2_stimulus_set,
                    percept_table)


# speedup vs baseline: 3.3118x; 2.3743x over previous
import functools

import jax
import jax.numpy as jnp
from jax import lax
from jax.experimental import pallas as pl
from jax.experimental.pallas import tpu as pltpu
from jax.experimental.pallas import tpu_sc as plsc

B = 16384
NSTIM = 21
NPAIR = NSTIM * NSTIM
NPAIR_PAD = 448
L = 16
NC, NS = 2, 16
NW = NC * NS
BPW = B // NW
CHUNK = 128
NCHUNKS = BPW // CHUNK
NGROUPS = CHUNK // L

_PAIRS = [(i, j) for i in range(8) for j in range(8) if j != i]


def _sqrt_f32(x):
    i = plsc.bitcast(x, jnp.int32)
    i = jnp.int32(0x5F3759DF) - jnp.right_shift(i, 1)
    y = plsc.bitcast(i, jnp.float32)
    for _ in range(3):
        y = y * (1.5 - 0.5 * x * y * y)
    return jnp.where(x > 0.0, x * y, 0.0)


def _splat(v, dtype=jnp.int32):
    return jnp.full((L,), v, dtype)


def _sc_body(g2_hbm, g8_hbm, tab_hbm, out1_hbm, out2_hbm,
             tab_v, s_v, idx2_v, idx8_v, out1_v, out2_v,
             sem_in2, sem_in8, sem_o1, sem_o2):
    wid = lax.axis_index("s") * NC + lax.axis_index("c")
    base = wid * BPW
    pltpu.sync_copy(tab_hbm, tab_v)

    def in_copy(c):
        cb = base + c * CHUNK
        b = c % 2
        return (pltpu.make_async_copy(
                    g2_hbm.at[:, pl.ds(cb, CHUNK)], idx2_v.at[b],
                    sem_in2.at[b]),
                pltpu.make_async_copy(
                    g8_hbm.at[:, pl.ds(cb, CHUNK)], idx8_v.at[b],
                    sem_in8.at[b]))

    def out_copy(c):
        cb = base + c * CHUNK
        b = c % 2
        return (pltpu.make_async_copy(
                    out1_v.at[b], out1_hbm.at[:, pl.ds(cb, CHUNK)],
                    sem_o1.at[b]),
                pltpu.make_async_copy(
                    out2_v.at[b], out2_hbm.at[:, pl.ds(cb, CHUNK)],
                    sem_o2.at[b]))

    for cp in in_copy(0):
        cp.start()

    def build_pairs(i, carry):
        p = jnp.minimum(i * L + lax.iota(jnp.int32, L), NPAIR - 1)
        q = jnp.right_shift(p * 3121, 16)
        r = p - q * NSTIM
        d2 = None
        for d in range(3):
            diff = (plsc.load_gather(tab_v, [_splat(d), q])
                    - plsc.load_gather(tab_v, [_splat(d), r]))
            d2 = diff * diff if d2 is None else d2 + diff * diff
        s_v[pl.ds(i * L, L)] = jnp.exp(-10.0 * _sqrt_f32(d2)) + 0.001
        return carry

    lax.fori_loop(0, NPAIR_PAD // L, build_pairs, 0)

    def make_group(i2_v, i8_v, o1_v, o2_v):
        def group(g, carry):
            sl = pl.ds(g * L, L)
            q = i2_v[0, sl] * NSTIM
            s1 = plsc.load_gather(s_v, [q + i2_v[1, sl]])
            s2 = plsc.load_gather(s_v, [q + i2_v[2, sl]])
            inv = 1.0 / (s1 + s2)
            o1_v[0, sl] = s1 * inv
            o1_v[1, sl] = s2 * inv
            q = i8_v[0, sl] * NSTIM
            s = [plsc.load_gather(s_v, [q + i8_v[j + 1, sl]])
                 for j in range(8)]
            tot = s[0]
            for j in range(1, 8):
                tot = tot + s[j]
            invt = 1.0 / tot
            a = [(s[i] * invt) / (tot - s[i]) for i in range(8)]
            for k, (i, j) in enumerate(_PAIRS):
                o2_v[k, sl] = a[i] * s[j]
            return carry
        return group

    for c in range(NCHUNKS):
        b = c % 2
        if c + 1 < NCHUNKS:
            for cp in in_copy(c + 1):
                cp.start()
        for cp in in_copy(c):
            cp.wait()
        if c >= 2:
            for cp in out_copy(c - 2):
                cp.wait()
        lax.fori_loop(0, NGROUPS,
                      make_group(idx2_v.at[b], idx8_v.at[b],
                                 out1_v.at[b], out2_v.at[b]),
                      0)
        for cp in out_copy(c):
            cp.start()

    for c in (NCHUNKS - 2, NCHUNKS - 1):
        for cp in out_copy(c):
            cp.wait()


@functools.cache
def _build():
    mesh = plsc.VectorSubcoreMesh(
        core_axis_name="c", subcore_axis_name="s",
        num_cores=NC, num_subcores=NS)
    return pl.kernel(
        _sc_body,
        out_type=(jax.ShapeDtypeStruct((2, B), jnp.float32),
                  jax.ShapeDtypeStruct((56, B), jnp.float32)),
        mesh=mesh,
        compiler_params=pltpu.CompilerParams(
            needs_layout_passes=False, use_tc_tiling_on_sc=True),
        scratch_types=[
            pltpu.VMEM((3, NSTIM), jnp.float32),
            pltpu.VMEM((NPAIR_PAD,), jnp.float32),
            pltpu.VMEM((2, 3, CHUNK), jnp.int32),
            pltpu.VMEM((2, 9, CHUNK), jnp.int32),
            pltpu.VMEM((2, 2, CHUNK), jnp.float32),
            pltpu.VMEM((2, 56, CHUNK), jnp.float32),
            pltpu.SemaphoreType.DMA((2,)),
            pltpu.SemaphoreType.DMA((2,)),
            pltpu.SemaphoreType.DMA((2,)),
            pltpu.SemaphoreType.DMA((2,)),
        ],
    )


def kernel(given2rank1_stimulus_set, given8rank2_stimulus_set, percept_table):
    o1t, o2t = _build()(given2rank1_stimulus_set.T,
                        given8rank2_stimulus_set.T,
                        percept_table.T)
    return (o1t.T, o2t.T)


# speedup vs baseline: 3.3178x; 1.0018x over previous
import functools

import jax
import jax.numpy as jnp
from jax import lax
from jax.experimental import pallas as pl
from jax.experimental.pallas import tpu as pltpu
from jax.experimental.pallas import tpu_sc as plsc

B = 16384
NSTIM = 21
NPAIR = NSTIM * NSTIM
NPAIR_PAD = 448
L = 16
NC, NS = 2, 16
NW = NC * NS
BPW = B // NW
HALF = BPW // 2
NGROUPS_HALF = HALF // L

_PAIRS = [(i, j) for i in range(8) for j in range(8) if j != i]


def _sqrt_f32(x):
    i = plsc.bitcast(x, jnp.int32)
    i = jnp.int32(0x5F3759DF) - jnp.right_shift(i, 1)
    y = plsc.bitcast(i, jnp.float32)
    for _ in range(3):
        y = y * (1.5 - 0.5 * x * y * y)
    return jnp.where(x > 0.0, x * y, 0.0)


def _splat(v, dtype=jnp.int32):
    return jnp.full((L,), v, dtype)


def _sc_body(g2_hbm, g8_hbm, tab_hbm, out1_hbm, out2_hbm,
             tab_v, s_v, idx2_v, idx8_v, out1_v, out2_v,
             sem_in2, sem_in8, sem_o1, sem_o2):
    wid = lax.axis_index("s") * NC + lax.axis_index("c")
    base = wid * BPW

    in2 = pltpu.make_async_copy(
        g2_hbm.at[:, pl.ds(base, BPW)], idx2_v, sem_in2)
    in8 = pltpu.make_async_copy(
        g8_hbm.at[:, pl.ds(base, BPW)], idx8_v, sem_in8)
    in2.start()
    in8.start()
    pltpu.sync_copy(tab_hbm, tab_v)

    def out_copy(h):
        hb = pl.ds(base + h * HALF, HALF)
        vb = pl.ds(h * HALF, HALF)
        return (pltpu.make_async_copy(
                    out1_v.at[:, vb], out1_hbm.at[:, hb], sem_o1),
                pltpu.make_async_copy(
                    out2_v.at[:, vb], out2_hbm.at[:, hb], sem_o2))

    def build_pairs(i, carry):
        p = jnp.minimum(i * L + lax.iota(jnp.int32, L), NPAIR - 1)
        q = jnp.right_shift(p * 3121, 16)
        r = p - q * NSTIM
        d2 = None
        for d in range(3):
            diff = (plsc.load_gather(tab_v, [_splat(d), q])
                    - plsc.load_gather(tab_v, [_splat(d), r]))
            d2 = diff * diff if d2 is None else d2 + diff * diff
        s_v[pl.ds(i * L, L)] = jnp.exp(-10.0 * _sqrt_f32(d2)) + 0.001
        return carry

    lax.fori_loop(0, NPAIR_PAD // L, build_pairs, 0)
    in2.wait()
    in8.wait()

    def group(g, carry):
        sl = pl.ds(g * L, L)
        q = idx2_v[0, sl] * NSTIM
        s1 = plsc.load_gather(s_v, [q + idx2_v[1, sl]])
        s2 = plsc.load_gather(s_v, [q + idx2_v[2, sl]])
        inv = 1.0 / (s1 + s2)
        out1_v[0, sl] = s1 * inv
        out1_v[1, sl] = s2 * inv
        q = idx8_v[0, sl] * NSTIM
        s = [plsc.load_gather(s_v, [q + idx8_v[j + 1, sl]])
             for j in range(8)]
        tot = s[0]
        for j in range(1, 8):
            tot = tot + s[j]
        invt = 1.0 / tot
        a = [(s[i] * invt) / (tot - s[i]) for i in range(8)]
        for k, (i, j) in enumerate(_PAIRS):
            out2_v[k, sl] = a[i] * s[j]
        return carry

    lax.fori_loop(0, NGROUPS_HALF, group, 0)
    first = out_copy(0)
    for cp in first:
        cp.start()
    lax.fori_loop(NGROUPS_HALF, 2 * NGROUPS_HALF, group, 0)
    second = out_copy(1)
    for cp in second:
        cp.start()
    for cp in first:
        cp.wait()
    for cp in second:
        cp.wait()


@functools.cache
def _build():
    mesh = plsc.VectorSubcoreMesh(
        core_axis_name="c", subcore_axis_name="s",
        num_cores=NC, num_subcores=NS)
    return pl.kernel(
        _sc_body,
        out_type=(jax.ShapeDtypeStruct((2, B), jnp.float32),
                  jax.ShapeDtypeStruct((56, B), jnp.float32)),
        mesh=mesh,
        compiler_params=pltpu.CompilerParams(
            needs_layout_passes=False, use_tc_tiling_on_sc=True),
        scratch_types=[
            pltpu.VMEM((3, NSTIM), jnp.float32),
            pltpu.VMEM((NPAIR_PAD,), jnp.float32),
            pltpu.VMEM((3, BPW), jnp.int32),
            pltpu.VMEM((9, BPW), jnp.int32),
            pltpu.VMEM((2, BPW), jnp.float32),
            pltpu.VMEM((56, BPW), jnp.float32),
            pltpu.SemaphoreType.DMA,
            pltpu.SemaphoreType.DMA,
            pltpu.SemaphoreType.DMA,
            pltpu.SemaphoreType.DMA,
        ],
    )


def kernel(given2rank1_stimulus_set, given8rank2_stimulus_set, percept_table):
    o1t, o2t = _build()(given2rank1_stimulus_set.T,
                        given8rank2_stimulus_set.T,
                        percept_table.T)
    return (o1t.T, o2t.T)
